# baseline (speedup 1.0000x reference)
import functools

import jax
import jax.numpy as jnp
from jax import lax
from jax.experimental import pallas as pl
from jax.experimental.pallas import tpu as pltpu
from jax.experimental.pallas import tpu_sc as plsc

_VOCAB = 100000
_EMBED = 128
_BATCH = 16384

_NC = 2
_NS = 16
_NW = _NC * _NS
_PER_W = _BATCH // _NW
_CHUNK = 128
_NCHUNK = _PER_W // _CHUNK
_NSLOT = 3
_LANES = 16


_TPAD = 17


def _dot_chunk(buf_a, buf_b, out_v, m_ref, out_base):
    lane = lax.iota(jnp.int32, _LANES)
    lo_mask = lane < (_LANES // 2)
    scat_base = lane * _TPAD

    _ILV = 8

    def group_body(g, _):
        row0 = g * _LANES
        for r4 in range(0, _LANES, _ILV):
            accs = [None] * _ILV
            for i in range(_EMBED // _LANES):
                for k in range(_ILV):
                    r = row0 + r4 + k
                    prod = (buf_a[r, pl.ds(i * _LANES, _LANES)]
                            * buf_b[r, pl.ds(i * _LANES, _LANES)])
                    accs[k] = prod if i == 0 else accs[k] + prod
            for k in range(_ILV):
                pairs = accs[k] + lax.rev(accs[k], (0,))
                plsc.store_scatter(m_ref, [scat_base + (r4 + k)], pairs, mask=lo_mask)
        sums = plsc.load_gather(m_ref, [lane])
        for j in range(1, _LANES // 2):
            sums = sums + plsc.load_gather(m_ref, [j * _TPAD + lane])
        out_v[pl.ds(out_base + row0, _LANES)] = sums
        return 0

    lax.fori_loop(0, _CHUNK // _LANES, group_body, 0)


def _sc_body(target_hbm, context_hbm, emb_in_hbm, emb_out_hbm, out_hbm,
             idx_a, idx_b, buf_a0, buf_a1, buf_a2, buf_b0, buf_b1, buf_b2,
             out_v, m_ref,
             sem_a0, sem_a1, sem_a2, sem_b0, sem_b1, sem_b2, sem_idx):
    wid = lax.axis_index("s") * _NC + lax.axis_index("c")
    base = wid * _PER_W

    ia = pltpu.async_copy(target_hbm.at[pl.ds(base, _PER_W)], idx_a, sem_idx)
    ib = pltpu.async_copy(context_hbm.at[pl.ds(base, _PER_W)], idx_b, sem_idx)
    ia.wait()
    ib.wait()

    bufs_a = (buf_a0, buf_a1, buf_a2)
    bufs_b = (buf_b0, buf_b1, buf_b2)
    sems_a = (sem_a0, sem_a1, sem_a2)
    sems_b = (sem_b0, sem_b1, sem_b2)

    def start(j):
        slot = j % _NSLOT
        ca = pltpu.async_copy(
            emb_in_hbm.at[idx_a.at[pl.ds(j * _CHUNK, _CHUNK)]], bufs_a[slot], sems_a[slot])
        cb = pltpu.async_copy(
            emb_out_hbm.at[idx_b.at[pl.ds(j * _CHUNK, _CHUNK)]], bufs_b[slot], sems_b[slot])
        return ca, cb

    inflight = {j: start(j) for j in range(min(_NSLOT, _NCHUNK))}
    for j in range(_NCHUNK):
        ca, cb = inflight.pop(j)
        ca.wait()
        cb.wait()
        slot = j % _NSLOT
        _dot_chunk(bufs_a[slot], bufs_b[slot], out_v, m_ref, j * _CHUNK)
        if j + _NSLOT < _NCHUNK:
            inflight[j + _NSLOT] = start(j + _NSLOT)

    pltpu.sync_copy(out_v, out_hbm.at[pl.ds(base, _PER_W)])


@functools.partial(jax.jit, static_argnames=())
def kernel(target, context, emb_in, emb_out):
    target = target.astype(jnp.int32)
    context = context.astype(jnp.int32)

    mesh = plsc.VectorSubcoreMesh(core_axis_name="c", subcore_axis_name="s")
    run = pl.kernel(
        _sc_body,
        out_type=jax.ShapeDtypeStruct((_BATCH,), jnp.float32),
        mesh=mesh,
        compiler_params=pltpu.CompilerParams(needs_layout_passes=False),
        scratch_types=[
            pltpu.VMEM((_PER_W,), jnp.int32),
            pltpu.VMEM((_PER_W,), jnp.int32),
            pltpu.VMEM((_CHUNK, _EMBED), jnp.float32),
            pltpu.VMEM((_CHUNK, _EMBED), jnp.float32),
            pltpu.VMEM((_CHUNK, _EMBED), jnp.float32),
            pltpu.VMEM((_CHUNK, _EMBED), jnp.float32),
            pltpu.VMEM((_CHUNK, _EMBED), jnp.float32),
            pltpu.VMEM((_CHUNK, _EMBED), jnp.float32),
            pltpu.VMEM((_PER_W,), jnp.float32),
            pltpu.VMEM((_CHUNK // _LANES * _LANES * _TPAD,), jnp.float32),
            pltpu.SemaphoreType.DMA,
            pltpu.SemaphoreType.DMA,
            pltpu.SemaphoreType.DMA,
            pltpu.SemaphoreType.DMA,
            pltpu.SemaphoreType.DMA,
            pltpu.SemaphoreType.DMA,
            pltpu.SemaphoreType.DMA,
        ],
    )
    return run(target, context, emb_in, emb_out)


# speedup vs baseline: 1.0311x; 1.0311x over previous
import functools

import jax
import jax.numpy as jnp
from jax import lax
from jax.experimental import pallas as pl
from jax.experimental.pallas import tpu as pltpu
from jax.experimental.pallas import tpu_sc as plsc

_VOCAB = 100000
_EMBED = 128
_BATCH = 16384

_NC = 2
_NS = 16
_NW = _NC * _NS
_PER_W = _BATCH // _NW
_CHUNK = 128
_NCHUNK = _PER_W // _CHUNK
_NSLOT = 3
_LANES = 16


_TPAD = 17


def _dot_chunk(buf_a, buf_b, out_v, m_ref, out_base):
    lane = lax.iota(jnp.int32, _LANES)
    lo_mask = lane < (_LANES // 2)
    scat_base = lane * _TPAD

    _ILV = 4

    def group_body(g, _):
        row0 = g * _LANES
        for r4 in range(0, _LANES, _ILV):
            acc0 = [None] * _ILV
            acc1 = [None] * _ILV
            for i in range(_EMBED // _LANES):
                for k in range(_ILV):
                    r = row0 + r4 + k
                    prod = (buf_a[r, pl.ds(i * _LANES, _LANES)]
                            * buf_b[r, pl.ds(i * _LANES, _LANES)])
                    if i % 2 == 0:
                        acc0[k] = prod if i == 0 else acc0[k] + prod
                    else:
                        acc1[k] = prod if i == 1 else acc1[k] + prod
            for k in range(_ILV):
                acc = acc0[k] + acc1[k]
                pairs = acc + lax.rev(acc, (0,))
                plsc.store_scatter(m_ref, [scat_base + (r4 + k)], pairs, mask=lo_mask)
        sums = plsc.load_gather(m_ref, [lane])
        for j in range(1, _LANES // 2):
            sums = sums + plsc.load_gather(m_ref, [j * _TPAD + lane])
        out_v[pl.ds(out_base + row0, _LANES)] = sums
        return 0

    lax.fori_loop(0, _CHUNK // _LANES, group_body, 0)


def _sc_body(target_hbm, context_hbm, emb_in_hbm, emb_out_hbm, out_hbm,
             idx_a, idx_b, buf_a0, buf_a1, buf_a2, buf_b0, buf_b1, buf_b2,
             out_v, m_ref,
             sem_a0, sem_a1, sem_a2, sem_b0, sem_b1, sem_b2, sem_idx):
    wid = lax.axis_index("s") * _NC + lax.axis_index("c")
    base = wid * _PER_W

    ia = pltpu.async_copy(target_hbm.at[pl.ds(base, _PER_W)], idx_a, sem_idx)
    ib = pltpu.async_copy(context_hbm.at[pl.ds(base, _PER_W)], idx_b, sem_idx)
    ia.wait()
    ib.wait()

    bufs_a = (buf_a0, buf_a1, buf_a2)
    bufs_b = (buf_b0, buf_b1, buf_b2)
    sems_a = (sem_a0, sem_a1, sem_a2)
    sems_b = (sem_b0, sem_b1, sem_b2)

    def start(j):
        slot = j % _NSLOT
        ca = pltpu.async_copy(
            emb_in_hbm.at[idx_a.at[pl.ds(j * _CHUNK, _CHUNK)]], bufs_a[slot], sems_a[slot])
        cb = pltpu.async_copy(
            emb_out_hbm.at[idx_b.at[pl.ds(j * _CHUNK, _CHUNK)]], bufs_b[slot], sems_b[slot])
        return ca, cb

    inflight = {j: start(j) for j in range(min(_NSLOT, _NCHUNK))}
    for j in range(_NCHUNK):
        ca, cb = inflight.pop(j)
        ca.wait()
        cb.wait()
        slot = j % _NSLOT
        _dot_chunk(bufs_a[slot], bufs_b[slot], out_v, m_ref, j * _CHUNK)
        if j + _NSLOT < _NCHUNK:
            inflight[j + _NSLOT] = start(j + _NSLOT)

    pltpu.sync_copy(out_v, out_hbm.at[pl.ds(base, _PER_W)])


@functools.partial(jax.jit, static_argnames=())
def kernel(target, context, emb_in, emb_out):
    target = target.astype(jnp.int32)
    context = context.astype(jnp.int32)

    mesh = plsc.VectorSubcoreMesh(core_axis_name="c", subcore_axis_name="s")
    run = pl.kernel(
        _sc_body,
        out_type=jax.ShapeDtypeStruct((_BATCH,), jnp.float32),
        mesh=mesh,
        compiler_params=pltpu.CompilerParams(needs_layout_passes=False),
        scratch_types=[
            pltpu.VMEM((_PER_W,), jnp.int32),
            pltpu.VMEM((_PER_W,), jnp.int32),
            pltpu.VMEM((_CHUNK, _EMBED), jnp.float32),
            pltpu.VMEM((_CHUNK, _EMBED), jnp.float32),
            pltpu.VMEM((_CHUNK, _EMBED), jnp.float32),
            pltpu.VMEM((_CHUNK, _EMBED), jnp.float32),
            pltpu.VMEM((_CHUNK, _EMBED), jnp.float32),
            pltpu.VMEM((_CHUNK, _EMBED), jnp.float32),
            pltpu.VMEM((_PER_W,), jnp.float32),
            pltpu.VMEM((_CHUNK // _LANES * _LANES * _TPAD,), jnp.float32),
            pltpu.SemaphoreType.DMA,
            pltpu.SemaphoreType.DMA,
            pltpu.SemaphoreType.DMA,
            pltpu.SemaphoreType.DMA,
            pltpu.SemaphoreType.DMA,
            pltpu.SemaphoreType.DMA,
            pltpu.SemaphoreType.DMA,
        ],
    )
    return run(target, context, emb_in, emb_out)


# speedup vs baseline: 1.0440x; 1.0124x over previous
import functools

import jax
import jax.numpy as jnp
from jax import lax
from jax.experimental import pallas as pl
from jax.experimental.pallas import tpu as pltpu
from jax.experimental.pallas import tpu_sc as plsc

_VOCAB = 100000
_EMBED = 128
_BATCH = 16384

_NC = 2
_NS = 16
_NW = _NC * _NS
_PER_W = _BATCH // _NW
_CHUNK = 128
_NCHUNK = _PER_W // _CHUNK
_NSLOT = 2
_LANES = 16


_TPAD = 17


def _dot_chunk(buf_a, buf_b, out_v, m_ref, out_base):
    lane = lax.iota(jnp.int32, _LANES)
    lo_mask = lane < (_LANES // 2)
    scat_base = lane * _TPAD

    _ILV = 4

    def group_body(g, _):
        row0 = g * _LANES
        for r4 in range(0, _LANES, _ILV):
            accs = [None] * _ILV
            for i in range(_EMBED // _LANES):
                for k in range(_ILV):
                    r = row0 + r4 + k
                    prod = (buf_a[r, pl.ds(i * _LANES, _LANES)]
                            * buf_b[r, pl.ds(i * _LANES, _LANES)])
                    accs[k] = prod if i == 0 else accs[k] + prod
            for k in range(_ILV):
                pairs = accs[k] + lax.rev(accs[k], (0,))
                plsc.store_scatter(m_ref, [scat_base + (r4 + k)], pairs, mask=lo_mask)
        sums = plsc.load_gather(m_ref, [lane])
        for j in range(1, _LANES // 2):
            sums = sums + plsc.load_gather(m_ref, [j * _TPAD + lane])
        out_v[pl.ds(out_base + row0, _LANES)] = sums
        return 0

    lax.fori_loop(0, _CHUNK // _LANES, group_body, 0)


def _sc_body(target_hbm, context_hbm, emb_in_hbm, emb_out_hbm, out_hbm,
             idx_v, buf_a0, buf_a1, buf_b0, buf_b1, out_v, m_ref,
             sem_a0, sem_a1, sem_b0, sem_b1):
    wid = lax.axis_index("s") * _NC + lax.axis_index("c")
    base = wid * _PER_W

    ia = pltpu.async_copy(target_hbm.at[pl.ds(base, _PER_W)],
                          idx_v.at[pl.ds(0, _PER_W)], sem_a0)
    ib = pltpu.async_copy(context_hbm.at[pl.ds(base, _PER_W)],
                          idx_v.at[pl.ds(_PER_W, _PER_W)], sem_b0)
    ia.wait()
    ib.wait()

    bufs_a = (buf_a0, buf_a1)
    bufs_b = (buf_b0, buf_b1)
    sems_a = (sem_a0, sem_a1)
    sems_b = (sem_b0, sem_b1)

    def start(j):
        slot = j % _NSLOT
        ca = pltpu.async_copy(
            emb_in_hbm.at[idx_v.at[pl.ds(j * _CHUNK, _CHUNK)]],
            bufs_a[slot], sems_a[slot])
        cb = pltpu.async_copy(
            emb_out_hbm.at[idx_v.at[pl.ds(_PER_W + j * _CHUNK, _CHUNK)]],
            bufs_b[slot], sems_b[slot])
        return ca, cb

    inflight = {j: start(j) for j in range(min(_NSLOT, _NCHUNK))}
    for j in range(_NCHUNK):
        ca, cb = inflight.pop(j)
        ca.wait()
        cb.wait()
        slot = j % _NSLOT
        _dot_chunk(bufs_a[slot], bufs_b[slot], out_v, m_ref, j * _CHUNK)
        if j + _NSLOT < _NCHUNK:
            inflight[j + _NSLOT] = start(j + _NSLOT)

    pltpu.sync_copy(out_v, out_hbm.at[pl.ds(base, _PER_W)])


@functools.partial(jax.jit, static_argnames=())
def kernel(target, context, emb_in, emb_out):
    target = target.astype(jnp.int32)
    context = context.astype(jnp.int32)

    mesh = plsc.VectorSubcoreMesh(core_axis_name="c", subcore_axis_name="s")
    run = pl.kernel(
        _sc_body,
        out_type=jax.ShapeDtypeStruct((_BATCH,), jnp.float32),
        mesh=mesh,
        compiler_params=pltpu.CompilerParams(needs_layout_passes=False),
        scratch_types=[
            pltpu.VMEM((2 * _PER_W,), jnp.int32),
            pltpu.VMEM((_CHUNK, _EMBED), jnp.float32),
            pltpu.VMEM((_CHUNK, _EMBED), jnp.float32),
            pltpu.VMEM((_CHUNK, _EMBED), jnp.float32),
            pltpu.VMEM((_CHUNK, _EMBED), jnp.float32),
            pltpu.VMEM((_PER_W,), jnp.float32),
            pltpu.VMEM((_LANES * _TPAD,), jnp.float32),
            pltpu.SemaphoreType.DMA,
            pltpu.SemaphoreType.DMA,
            pltpu.SemaphoreType.DMA,
            pltpu.SemaphoreType.DMA,
        ],
    )
    return run(target, context, emb_in, emb_out)
